# baseline (speedup 1.0000x reference)
import jax
import jax.numpy as jnp
from jax.experimental import pallas as pl
from jax.experimental.pallas import tpu as pltpu

_SPARSE = tuple(range(8, 16))
_NSP = len(_SPARSE)
_K = 25
_PADC = 32
_BIG = 3.0e38


def _pick_block(n):
    best = None
    for br in range(8, min(n, 8192) + 1, 8):
        if n % br == 0 and (best is None or abs(br - 2048) < abs(best - 2048)):
            best = br
    return best if best is not None else n


def _labels_of(logits):
    rowmax = jnp.max(logits, axis=1, keepdims=True)
    cols = jax.lax.broadcasted_iota(jnp.int32, logits.shape, 1)
    return jnp.min(jnp.where(logits == rowmax, cols, _PADC), axis=1, keepdims=True)


def _head_body(feat_ref, w_ref, b_ref, logits_ref, lab_ref, nc):
    logits = jnp.dot(feat_ref[...], w_ref[...], preferred_element_type=jnp.float32)
    logits = logits + b_ref[...]
    logits_ref[...] = logits[:, :nc]
    lab_ref[...] = _labels_of(logits)


def _mask_body(x_ref, y_ref, z_ref, lab_ref, reset_ref, kk):
    x = x_ref[...]
    y = y_ref[...]
    z = z_ref[...]
    lab = lab_ref[...]
    d2s = []
    masks = []
    ts = []
    for c in _SPARSE:
        m = lab == c
        cnt = jnp.sum(jnp.where(m, 1.0, 0.0))
        safe = jnp.maximum(cnt, 1.0)
        cx = jnp.sum(jnp.where(m, x, 0.0)) / safe
        cy = jnp.sum(jnp.where(m, y, 0.0)) / safe
        cz = jnp.sum(jnp.where(m, z, 0.0)) / safe
        d2 = (x - cx) ** 2 + (y - cy) ** 2 + (z - cz) ** 2
        d2s.append(d2)
        masks.append(m)
        ts.append(jnp.min(d2))
    for _ in range(kk - 1):
        ts = [jnp.min(jnp.where(d2s[i] > ts[i], d2s[i], _BIG)) for i in range(_NSP)]
    reset = jnp.zeros(x.shape, jnp.bool_)
    for i in range(_NSP):
        reset = reset | (masks[i] & (d2s[i] > ts[i]))
    reset_ref[...] = jnp.where(reset, 1.0, 0.0)


def kernel(feat, coord, W, b):
    n, c_in = feat.shape
    nc = W.shape[1]
    br = _pick_block(n)
    nb = n // br

    w_pad = jnp.pad(W.astype(jnp.float32), ((0, 0), (0, _PADC - nc)))
    b_pad = jnp.pad(
        b.astype(jnp.float32).reshape(1, nc),
        ((0, 0), (0, _PADC - nc)),
        constant_values=-1.0e30,
    )

    logits20, labels = pl.pallas_call(
        lambda fr, wr, br_, lg, lb: _head_body(fr, wr, br_, lg, lb, nc),
        grid=(nb,),
        in_specs=[
            pl.BlockSpec((br, c_in), lambda i: (i, 0)),
            pl.BlockSpec((c_in, _PADC), lambda i: (0, 0)),
            pl.BlockSpec((1, _PADC), lambda i: (0, 0)),
        ],
        out_specs=[
            pl.BlockSpec((br, nc), lambda i: (i, 0)),
            pl.BlockSpec((br, 1), lambda i: (i, 0)),
        ],
        out_shape=[
            jax.ShapeDtypeStruct((n, nc), jnp.float32),
            jax.ShapeDtypeStruct((n, 1), jnp.int32),
        ],
    )(feat, w_pad, b_pad)

    ntot = ((n + 1023) // 1024) * 1024
    rr = ntot // 128
    pad_f = jnp.full((ntot - n,), 1.0e6, jnp.float32)
    planes = [
        jnp.concatenate([coord[:, d].astype(jnp.float32), pad_f]).reshape(rr, 128)
        for d in range(3)
    ]
    lab_plane = jnp.concatenate(
        [labels.reshape(n), jnp.full((ntot - n,), -1, jnp.int32)]
    ).reshape(rr, 128)

    kk = min(_K, n)
    reset_plane = pl.pallas_call(
        lambda xr, yr, zr, lr, rr_: _mask_body(xr, yr, zr, lr, rr_, kk),
        in_specs=[pl.BlockSpec(memory_space=pltpu.VMEM)] * 4,
        out_specs=pl.BlockSpec(memory_space=pltpu.VMEM),
        out_shape=jax.ShapeDtypeStruct((rr, 128), jnp.float32),
    )(planes[0], planes[1], planes[2], lab_plane)

    reset_col = reset_plane.reshape(ntot)[:n].reshape(n, 1)

    def _apply_body(logits_ref, reset_ref, out_ref):
        logits = logits_ref[...]
        cols = jax.lax.broadcasted_iota(jnp.int32, logits.shape, 1)
        target = jnp.where(cols == 1, jnp.float32(10.0), jnp.float32(0.0))
        resetb = reset_ref[...] != 0.0
        out_ref[...] = jnp.where(resetb, target, logits)

    out = pl.pallas_call(
        _apply_body,
        grid=(nb,),
        in_specs=[
            pl.BlockSpec((br, nc), lambda i: (i, 0)),
            pl.BlockSpec((br, 1), lambda i: (i, 0)),
        ],
        out_specs=pl.BlockSpec((br, nc), lambda i: (i, 0)),
        out_shape=jax.ShapeDtypeStruct((n, nc), jnp.float32),
    )(logits20, reset_col)

    return out
